# Initial kernel scaffold; baseline (speedup 1.0000x reference)
import jax
import jax.numpy as jnp
from jax.experimental import pallas as pl


def kernel(tensor, i, j):
    raise NotImplementedError("write your pallas kernel here")


# speedup vs baseline: 2.7648x; 2.7648x over previous
import jax
import jax.numpy as jnp
from jax import lax
from jax.experimental import pallas as pl
from jax.experimental.pallas import tpu as pltpu
from jax.experimental.pallas import tpu_sc as plsc

_OUT = 192
_H = 224
_W = 224
_NC = 2
_NS = 16
_NW = _NC * _NS
_LANES = 16
_GROUPS = _OUT // _LANES


def _sc_body(in_hbm, i_hbm, j_hbm, out_hbm, ij_buf, idx_buf, in_buf, out_buf, sem):
    VC = in_hbm.shape[0] // _H
    C = VC // 16
    per_w = VC // _NW
    wid = lax.axis_index("s") * _NC + lax.axis_index("c")
    pltpu.sync_copy(i_hbm, ij_buf.at[0])
    pltpu.sync_copy(j_hbm, ij_buf.at[1])
    lane = lax.iota(jnp.int32, _LANES)

    def pair_body(p, carry):
        pair = wid * per_w + p
        n = pair // C
        nvec = jnp.full((_LANES,), n, jnp.int32)
        ivec = plsc.load_gather(ij_buf, [jnp.zeros((_LANES,), jnp.int32), nvec])
        jvec = plsc.load_gather(ij_buf, [jnp.ones((_LANES,), jnp.int32), nvec])
        rowbase = pair * _H + ivec + lane

        def idx_body(g, c2):
            idx_buf[pl.ds(g * _LANES, _LANES)] = rowbase + g * _LANES
            return c2

        lax.fori_loop(0, _GROUPS, idx_body, 0)
        pltpu.async_copy(in_hbm.at[idx_buf], in_buf, sem).wait()

        def g_body(g, c2):
            row = g // _GROUPS
            colbase = (g % _GROUPS) * _LANES
            cols = colbase + jvec + lane
            v = plsc.load_gather(in_buf, [jnp.full((_LANES,), row, jnp.int32), cols])
            out_buf[row, pl.ds(colbase, _LANES)] = v
            return c2

        lax.fori_loop(0, _OUT * _GROUPS, g_body, 0)
        pltpu.sync_copy(out_buf, out_hbm.at[pl.ds(pair * _OUT, _OUT)])
        return carry

    lax.fori_loop(0, per_w, pair_body, 0)


def kernel(tensor, i, j):
    N, C, H, W = tensor.shape
    mesh = plsc.VectorSubcoreMesh(core_axis_name="c", subcore_axis_name="s")
    run = pl.kernel(
        _sc_body,
        out_type=jax.ShapeDtypeStruct((N * C * _OUT, _OUT), tensor.dtype),
        mesh=mesh,
        scratch_types=[
            pltpu.VMEM((2, _LANES), jnp.int32),
            pltpu.VMEM((_OUT,), jnp.int32),
            pltpu.VMEM((_OUT, _W), jnp.float32),
            pltpu.VMEM((_OUT, _OUT), jnp.float32),
            pltpu.SemaphoreType.DMA,
        ],
        compiler_params=pltpu.CompilerParams(
            use_tc_tiling_on_sc=False, needs_layout_passes=False),
    )
    out = run(tensor.reshape(N * C * H, W), i, j)
    return out.reshape(N, C, _OUT, _OUT)


# speedup vs baseline: 2.8855x; 1.0436x over previous
import jax
import jax.numpy as jnp
from jax import lax
from jax.experimental import pallas as pl
from jax.experimental.pallas import tpu as pltpu
from jax.experimental.pallas import tpu_sc as plsc

_OUT = 192
_H = 224
_W = 224
_NC = 2
_NS = 16
_NW = _NC * _NS
_LANES = 16
_GROUPS = _OUT // _LANES


def _sc_body(in_hbm, i_hbm, j_hbm, out_hbm, ij_buf, idx_buf, in_buf, out_buf, sem):
    VC = in_hbm.shape[0] // _H
    C = VC // 16
    per_w = VC // _NW
    wid = lax.axis_index("s") * _NC + lax.axis_index("c")
    pltpu.sync_copy(i_hbm, ij_buf.at[0])
    pltpu.sync_copy(j_hbm, ij_buf.at[1])
    lane = lax.iota(jnp.int32, _LANES)

    def pair_body(p, carry):
        pair = wid * per_w + p
        n = pair // C
        nvec = jnp.full((_LANES,), n, jnp.int32)
        ivec = plsc.load_gather(ij_buf, [jnp.zeros((_LANES,), jnp.int32), nvec])
        jvec = plsc.load_gather(ij_buf, [jnp.ones((_LANES,), jnp.int32), nvec])
        rowbase = pair * _H + ivec + lane

        def idx_body(g, c2):
            idx_buf[pl.ds(g * _LANES, _LANES)] = rowbase + g * _LANES
            return c2

        lax.fori_loop(0, _GROUPS, idx_body, 0)
        pltpu.async_copy(in_hbm.at[idx_buf], in_buf, sem).wait()

        cols = [g * _LANES + jvec + lane for g in range(_GROUPS)]

        def row_body(row, c2):
            rowvec = jnp.full((_LANES,), row, jnp.int32)
            for g in range(_GROUPS):
                v = plsc.load_gather(in_buf, [rowvec, cols[g]])
                out_buf[row, pl.ds(g * _LANES, _LANES)] = v
            return c2

        lax.fori_loop(0, _OUT, row_body, 0)
        pltpu.sync_copy(out_buf, out_hbm.at[pl.ds(pair * _OUT, _OUT)])
        return carry

    lax.fori_loop(0, per_w, pair_body, 0)


def kernel(tensor, i, j):
    N, C, H, W = tensor.shape
    mesh = plsc.VectorSubcoreMesh(core_axis_name="c", subcore_axis_name="s")
    run = pl.kernel(
        _sc_body,
        out_type=jax.ShapeDtypeStruct((N * C * _OUT, _OUT), tensor.dtype),
        mesh=mesh,
        scratch_types=[
            pltpu.VMEM((2, _LANES), jnp.int32),
            pltpu.VMEM((_OUT,), jnp.int32),
            pltpu.VMEM((_OUT, _W), jnp.float32),
            pltpu.VMEM((_OUT, _OUT), jnp.float32),
            pltpu.SemaphoreType.DMA,
        ],
        compiler_params=pltpu.CompilerParams(
            use_tc_tiling_on_sc=False, needs_layout_passes=False),
    )
    out = run(tensor.reshape(N * C * H, W), i, j)
    return out.reshape(N, C, _OUT, _OUT)


# speedup vs baseline: 5.5221x; 1.9138x over previous
import jax
import jax.numpy as jnp
from jax import lax
from jax.experimental import pallas as pl
from jax.experimental.pallas import tpu as pltpu
from jax.experimental.pallas import tpu_sc as plsc

_OUT = 192
_H = 224
_W = 224
_WIN = 200
_NC = 2
_NS = 16
_NW = _NC * _NS
_LANES = 16
_GROUPS = _OUT // _LANES


def _sc_body(in_hbm, i_hbm, j_hbm, out_hbm, ij_buf, in_buf, out_buf):
    N, C, H, W = in_hbm.shape
    per_w = (N * C) // _NW
    wid = lax.axis_index("s") * _NC + lax.axis_index("c")
    pltpu.sync_copy(i_hbm, ij_buf.at[0])
    pltpu.sync_copy(j_hbm, ij_buf.at[1])
    lane = lax.iota(jnp.int32, _LANES)

    def pair_body(p, carry):
        pair = wid * per_w + p
        n = pair // C
        cc = pair % C
        sel = lane == n
        i0 = jnp.sum(jnp.where(sel, ij_buf[0, :], 0))
        j0 = jnp.sum(jnp.where(sel, ij_buf[1, :], 0))
        ibase = pl.multiple_of((i0 // 8) * 8, 8)
        rsub = i0 - ibase
        pltpu.sync_copy(in_hbm.at[n, cc, pl.ds(ibase, _WIN), :], in_buf)

        cols = [g * _LANES + j0 + lane for g in range(_GROUPS)]

        def row_body(row, c2):
            rowvec = jnp.full((_LANES,), row + rsub, jnp.int32)
            for g in range(_GROUPS):
                v = plsc.load_gather(in_buf, [rowvec, cols[g]])
                out_buf[row, pl.ds(g * _LANES, _LANES)] = v
            return c2

        lax.fori_loop(0, _OUT, row_body, 0)
        pltpu.sync_copy(out_buf, out_hbm.at[n, cc])
        return carry

    lax.fori_loop(0, per_w, pair_body, 0)


def kernel(tensor, i, j):
    N, C, H, W = tensor.shape
    mesh = plsc.VectorSubcoreMesh(core_axis_name="c", subcore_axis_name="s")
    run = pl.kernel(
        _sc_body,
        out_type=jax.ShapeDtypeStruct((N, C, _OUT, _OUT), tensor.dtype),
        mesh=mesh,
        scratch_types=[
            pltpu.VMEM((2, _LANES), jnp.int32),
            pltpu.VMEM((_WIN, _W), jnp.float32),
            pltpu.VMEM((_OUT, _OUT), jnp.float32),
        ],
        compiler_params=pltpu.CompilerParams(
            use_tc_tiling_on_sc=True, needs_layout_passes=False),
    )
    return run(tensor, i, j)


# speedup vs baseline: 11.3677x; 2.0586x over previous
import jax
import jax.numpy as jnp
from jax import lax
from jax.experimental import pallas as pl
from jax.experimental.pallas import tpu as pltpu
from jax.experimental.pallas import tpu_sc as plsc

_OUT = 192
_H = 224
_W = 224
_WIN = 200
_NC = 2
_NS = 16
_NW = _NC * _NS
_LANES = 16
_GROUPS = _OUT // _LANES


def _sc_body(in_hbm, i_hbm, j_hbm, out_hbm, ij_buf, in_buf, out_buf):
    N, C, H, W = in_hbm.shape
    per_w = (N * C) // _NW
    wid = lax.axis_index("s") * _NC + lax.axis_index("c")
    pltpu.sync_copy(i_hbm, ij_buf.at[0])
    pltpu.sync_copy(j_hbm, ij_buf.at[1])
    lane = lax.iota(jnp.int32, _LANES)

    def pair_body(p, carry):
        pair = wid * per_w + p
        n = pair // C
        cc = pair % C
        sel = lane == n
        i0 = jnp.sum(jnp.where(sel, ij_buf[0, :], 0))
        j0 = jnp.sum(jnp.where(sel, ij_buf[1, :], 0))
        ibase = pl.multiple_of((i0 // 8) * 8, 8)
        rsub = i0 - ibase
        pltpu.sync_copy(in_hbm.at[n, cc, pl.ds(ibase, _WIN), :], in_buf)

        cols = [g * _LANES + j0 + lane for g in range(_GROUPS)]

        @plsc.parallel_loop(0, _OUT, step=1, unroll=2)
        def row_body(row):
            rowvec = jnp.full((_LANES,), row + rsub, jnp.int32)
            for g in range(_GROUPS):
                v = plsc.load_gather(in_buf, [rowvec, cols[g]])
                out_buf[row, pl.ds(g * _LANES, _LANES)] = v
        pltpu.sync_copy(out_buf, out_hbm.at[n, cc])
        return carry

    lax.fori_loop(0, per_w, pair_body, 0)


def kernel(tensor, i, j):
    N, C, H, W = tensor.shape
    mesh = plsc.VectorSubcoreMesh(core_axis_name="c", subcore_axis_name="s")
    run = pl.kernel(
        _sc_body,
        out_type=jax.ShapeDtypeStruct((N, C, _OUT, _OUT), tensor.dtype),
        mesh=mesh,
        scratch_types=[
            pltpu.VMEM((2, _LANES), jnp.int32),
            pltpu.VMEM((_WIN, _W), jnp.float32),
            pltpu.VMEM((_OUT, _OUT), jnp.float32),
        ],
        compiler_params=pltpu.CompilerParams(
            use_tc_tiling_on_sc=True, needs_layout_passes=False),
    )
    return run(tensor, i, j)
